# baseline (speedup 1.0000x reference)
import jax
import jax.numpy as jnp
from jax import lax
from jax.experimental import pallas as pl
from jax.experimental.pallas import tpu as pltpu
from jax.experimental.pallas import tpu_sc as plsc

_BATCH = 16384
_D = 3200
_NUM_GROUPS = 10
_NUM_SLICES = 5
_CHUNK = 32
_GROUP_W = _NUM_SLICES * _CHUNK

_info = plsc.get_sparse_core_info()
_NC = _info.num_cores
_NS = _info.num_subcores
_NW = _NC * _NS
_RPW = _BATCH // _NW


def _body(in_hbm, *rest):
    outs = rest[:_NUM_GROUPS]
    sem = rest[_NUM_GROUPS]
    wid = lax.axis_index("s") * _NC + lax.axis_index("c")
    row0 = wid * _RPW
    copies = []
    for g in range(_NUM_GROUPS):
        for j in range(_NUM_SLICES):
            src_col = (j * _NUM_GROUPS + g) * _CHUNK
            copies.append(
                pltpu.make_async_copy(
                    in_hbm.at[pl.ds(row0, _RPW), pl.ds(src_col, _CHUNK)],
                    outs[g].at[pl.ds(row0, _RPW), pl.ds(j * _CHUNK, _CHUNK)],
                    sem,
                )
            )
    for c in copies:
        c.start()
    for c in copies:
        c.wait()


def kernel(input_tensor):
    out_type = [
        jax.ShapeDtypeStruct((_BATCH, _GROUP_W), jnp.float32)
    ] * _NUM_GROUPS
    f = pl.kernel(
        _body,
        out_type=out_type,
        mesh=plsc.VectorSubcoreMesh(core_axis_name="c", subcore_axis_name="s"),
        scratch_types=[pltpu.SemaphoreType.DMA],
        compiler_params=pltpu.CompilerParams(use_tc_tiling_on_sc=False),
    )
    return tuple(f(input_tensor))


# speedup vs baseline: 6.9214x; 6.9214x over previous
import numpy as np

import jax
import jax.numpy as jnp
from jax import lax
from jax.experimental import pallas as pl
from jax.experimental.pallas import tpu as pltpu
from jax.experimental.pallas import tpu_sc as plsc

_BATCH = 16384
_D = 3200
_NUM_GROUPS = 10
_NUM_SLICES = 5
_CHUNK = 32
_GROUP_W = _NUM_SLICES * _CHUNK
_BLOCKS_PER_ROW = _D // _CHUNK

_info = plsc.get_sparse_core_info()
_NC = _info.num_cores
_NS = _info.num_subcores
_NW = _NC * _NS
_RPW = _BATCH // _NW
_GROWS = _RPW * _NUM_SLICES
_IDX_W = 128
_NSLAB = _GROWS // _IDX_W


def _build_indices():
    i = np.arange(_GROWS, dtype=np.int64)
    base = (i // _NUM_SLICES) * _BLOCKS_PER_ROW + (i % _NUM_SLICES) * _NUM_GROUPS
    w = np.arange(_NW, dtype=np.int64)[:, None, None]
    g = np.arange(_NUM_GROUPS, dtype=np.int64)[None, :, None]
    idx = w * (_RPW * _BLOCKS_PER_ROW) + g + base[None, None, :]
    return idx.astype(np.int32).reshape(_NW, _NUM_GROUPS, _NSLAB, _IDX_W)


_IDX_NP = _build_indices()


def _body(tbl, idx_hbm, *rest):
    outs = rest[:_NUM_GROUPS]
    idx_v, dst_v, gsem, wsem = rest[_NUM_GROUPS:]
    wid = lax.axis_index("s") * _NC + lax.axis_index("c")
    row0 = wid * _RPW

    pltpu.make_async_copy(idx_hbm.at[wid], idx_v, wsem).start()
    pltpu.make_async_copy(idx_hbm.at[wid], idx_v, wsem).wait()

    for g in range(_NUM_GROUPS):
        copies = []
        for k in range(_NSLAB):
            copies.append(
                pltpu.make_async_copy(
                    tbl.at[idx_v.at[g, k]],
                    dst_v.at[pl.ds(k * _IDX_W, _IDX_W)],
                    gsem,
                )
            )
        for c in copies:
            c.start()
        for c in copies:
            c.wait()

        w = pltpu.make_async_copy(
            dst_v, outs[g].at[pl.ds(row0 * _NUM_SLICES, _GROWS)], wsem
        )
        w.start()
        w.wait()


def kernel(input_tensor):
    tbl = input_tensor.reshape(_BATCH * _BLOCKS_PER_ROW, _CHUNK)
    idx = jnp.asarray(_IDX_NP)
    out_type = [
        jax.ShapeDtypeStruct((_BATCH * _NUM_SLICES, _CHUNK), jnp.float32)
    ] * _NUM_GROUPS
    f = pl.kernel(
        _body,
        out_type=out_type,
        mesh=plsc.VectorSubcoreMesh(core_axis_name="c", subcore_axis_name="s"),
        scratch_types=[
            pltpu.VMEM((_NUM_GROUPS, _NSLAB, _IDX_W), jnp.int32),
            pltpu.VMEM((_GROWS, _CHUNK), jnp.float32),
            pltpu.SemaphoreType.DMA,
            pltpu.SemaphoreType.DMA,
        ],
        compiler_params=pltpu.CompilerParams(use_tc_tiling_on_sc=False),
    )
    outs = f(tbl, idx)
    return tuple(o.reshape(_BATCH, _GROUP_W) for o in outs)
